# baseline (speedup 1.0000x reference)
import functools

import jax
import jax.numpy as jnp
from jax import lax
from jax.experimental import pallas as pl
from jax.experimental.pallas import tpu as pltpu
from jax.experimental.pallas import tpu_sc as plsc

D = 64
CTX = 10
NNEG = 20

NC = 2
NS = 16
NW = NC * NS
L = 16


def _sc_gather(i_emb, o_emb, ctx_ids, tgt_ids, neg_ids):
  B = tgt_ids.shape[0]
  spw = B // NW

  mesh = plsc.VectorSubcoreMesh(core_axis_name="c", subcore_axis_name="s")

  @functools.partial(
      pl.kernel,
      mesh=mesh,
      out_type=[
          jax.ShapeDtypeStruct((CTX // 2 * B, 2 * D), jnp.float32),
          jax.ShapeDtypeStruct((B, D), jnp.float32),
          jax.ShapeDtypeStruct((NNEG // 2 * B, 2 * D), jnp.float32),
      ],
      scratch_types=[
          pltpu.VMEM((spw * CTX,), jnp.int32),
          pltpu.VMEM((spw * NNEG,), jnp.int32),
          pltpu.VMEM((spw,), jnp.int32),
          pltpu.VMEM((spw,), jnp.int32),
          pltpu.VMEM((spw, D), jnp.float32),
          pltpu.VMEM((spw, D), jnp.float32),
          pltpu.SemaphoreType.DMA,
          pltpu.SemaphoreType.DMA,
      ],
      compiler_params=pltpu.CompilerParams(use_tc_tiling_on_sc=False,
                                           needs_layout_passes=False),
  )
  def gather_k(i_emb_h, o_emb_h, ctx_ids_h, tgt_ids_h, neg_ids_h,
               ctx_out, tgt_out, neg_out,
               cv, nv, i0, i1, buf0, buf1, sem0, sem1):
    wid = lax.axis_index("s") * NC + lax.axis_index("c")
    base = wid * spw
    ibufs = (i0, i1)
    bufs = (buf0, buf1)
    sems = (sem0, sem1)

    pltpu.sync_copy(ctx_ids_h.at[pl.ds(base * CTX, spw * CTX)], cv)
    pltpu.sync_copy(neg_ids_h.at[pl.ds(base * NNEG, spw * NNEG)], nv)

    def build_idx(src, nslot, j, ibuf):
      for k in range(spw // L):
        r = (lax.iota(jnp.int32, L) + (L * k)) * nslot + j
        ibuf[pl.ds(L * k, L)] = plsc.load_gather(src, [r])

    def run(tab_h, n_slots, build, scatter):
      def fire(j, slot):
        build(j, ibufs[slot])
        pltpu.async_copy(tab_h.at[ibufs[slot]], bufs[slot], sems[slot])

      def drain(j, slot):
        pltpu.make_async_copy(tab_h.at[ibufs[slot]], bufs[slot],
                              sems[slot]).wait()
        scatter(j, bufs[slot])

      if n_slots == 1:
        fire(0, 0)
        drain(0, 0)
        return

      fire(0, 0)

      def body(k, carry):
        j0 = k * 2
        fire(j0 + 1, 1)
        drain(j0, 0)

        @pl.when(k < n_slots // 2 - 1)
        def _():
          fire(j0 + 2, 0)

        drain(j0 + 1, 1)
        return carry

      lax.fori_loop(0, n_slots // 2, body, 0)

    def scatter_pair(out_h):
      def scatter(j, buf):
        row0 = (j // 2) * B + base
        col0 = (j % 2) * D
        pltpu.sync_copy(buf, out_h.at[pl.ds(row0, spw), pl.ds(col0, D)])
      return scatter

    run(i_emb_h, CTX,
        lambda j, ibuf: build_idx(cv, CTX, j, ibuf),
        scatter_pair(ctx_out))
    run(o_emb_h, 1,
        lambda j, ibuf: pltpu.sync_copy(tgt_ids_h.at[pl.ds(base, spw)],
                                        ibuf),
        lambda j, buf: pltpu.sync_copy(buf, tgt_out.at[pl.ds(base, spw)]))
    run(o_emb_h, NNEG,
        lambda j, ibuf: build_idx(nv, NNEG, j, ibuf),
        scatter_pair(neg_out))

  return gather_k(i_emb, o_emb, ctx_ids, tgt_ids, neg_ids)


def _dense_body(ctx_ref, tgt_ref, neg_ref, W1_ref, b1_ref, W2_ref, b2_ref,
                out_ref):
  f32 = jnp.float32

  hp = jnp.dot(ctx_ref[0], W1_ref[pl.ds(0, 2 * D), :],
               preferred_element_type=f32)
  for jp in range(1, CTX // 2):
    hp = hp + jnp.dot(ctx_ref[jp], W1_ref[pl.ds(jp * 2 * D, 2 * D), :],
                      preferred_element_type=f32)
  h = jnp.tanh(hp + b1_ref[...])
  logits = jnp.dot(h, W2_ref[...],
                   preferred_element_type=f32) + b2_ref[...]
  a = jax.nn.softmax(logits, axis=-1)

  attn128 = jnp.zeros(hp.shape[:1] + (2 * D,), f32)
  lane128 = lax.broadcasted_iota(jnp.int32, hp.shape[:1] + (2 * D,), 1)
  for jp in range(CTX // 2):
    aw = jnp.where(lane128 < D, a[:, 2 * jp:2 * jp + 1],
                   a[:, 2 * jp + 1:2 * jp + 2])
    attn128 = attn128 + aw * ctx_ref[jp]
  attn = attn128[:, 0:D] + attn128[:, D:2 * D]

  pos_dot = jnp.sum(tgt_ref[...] * attn, axis=1, keepdims=True)

  cols128 = lax.broadcasted_iota(jnp.int32, (2 * D, 2), 0)
  sel = lax.broadcasted_iota(jnp.int32, (2 * D, 2), 1)
  seg2 = jnp.where(sel == 0, cols128 < D, cols128 >= D).astype(f32)
  attn2 = jnp.concatenate([attn, attn], axis=1)
  nds = []
  for k in range(NNEG // 2):
    prod = neg_ref[k] * attn2
    nds.append(jnp.concatenate(
        [jnp.sum(prod[:, 0:D], axis=1, keepdims=True),
         jnp.sum(prod[:, D:2 * D], axis=1, keepdims=True)], axis=1))
  all_dots = jnp.concatenate([pos_dot] + [-n for n in nds], axis=1)

  acc = jnp.sum(jnp.log(jax.nn.sigmoid(all_dots)))

  @pl.when(pl.program_id(0) == 0)
  def _():
    out_ref[0, 0] = 0.0

  out_ref[0, 0] += acc


def kernel(target_wids, context_wids, neg_wids, i_emb, o_emb, W1, b1, W2, b2):
  B = target_wids.shape[0]
  ctx_ids = context_wids.astype(jnp.int32).reshape(-1)
  tgt_ids = target_wids.astype(jnp.int32)
  neg_ids = neg_wids.astype(jnp.int32).reshape(-1)

  ctx_pair, tgt, neg_pair = _sc_gather(i_emb, o_emb, ctx_ids, tgt_ids,
                                       neg_ids)
  ctx3 = ctx_pair.reshape(CTX // 2, B, 2 * D)
  neg3 = neg_pair.reshape(NNEG // 2, B, 2 * D)

  bB = 1024
  grid = B // bB
  loss = pl.pallas_call(
      _dense_body,
      grid=(grid,),
      in_specs=[
          pl.BlockSpec((CTX // 2, bB, 2 * D), lambda i: (0, i, 0)),
          pl.BlockSpec((bB, D), lambda i: (i, 0)),
          pl.BlockSpec((NNEG // 2, bB, 2 * D), lambda i: (0, i, 0)),
          pl.BlockSpec((CTX * D, 50), lambda i: (0, 0)),
          pl.BlockSpec((1, 50), lambda i: (0, 0)),
          pl.BlockSpec((50, CTX), lambda i: (0, 0)),
          pl.BlockSpec((1, CTX), lambda i: (0, 0)),
      ],
      out_specs=pl.BlockSpec((1, 1), lambda i: (0, 0),
                             memory_space=pltpu.SMEM),
      out_shape=jax.ShapeDtypeStruct((1, 1), jnp.float32),
  )(ctx3, tgt, neg3, W1, b1.reshape(1, 50), W2, b2.reshape(1, CTX))

  return -loss[0, 0]


# speedup vs baseline: 1.2146x; 1.2146x over previous
import functools

import jax
import jax.numpy as jnp
from jax import lax
from jax.experimental import pallas as pl
from jax.experimental.pallas import tpu as pltpu
from jax.experimental.pallas import tpu_sc as plsc

D = 64
CTX = 10
NNEG = 20

NC = 2
NS = 16
NW = NC * NS
L = 16


def _sc_gather(i_emb, o_emb, ctx_ids, tgt_ids, neg_ids):
  B = tgt_ids.shape[0]
  spw = B // NW

  mesh = plsc.VectorSubcoreMesh(core_axis_name="c", subcore_axis_name="s")

  @functools.partial(
      pl.kernel,
      mesh=mesh,
      out_type=[
          jax.ShapeDtypeStruct((CTX // 2 * B, 2 * D), jnp.float32),
          jax.ShapeDtypeStruct((B, D), jnp.float32),
          jax.ShapeDtypeStruct((NNEG // 2 * B, 2 * D), jnp.float32),
      ],
      scratch_types=[
          pltpu.VMEM((spw,), jnp.int32),
          pltpu.VMEM((spw,), jnp.int32),
          pltpu.VMEM((spw, D), jnp.float32),
          pltpu.VMEM((spw, D), jnp.float32),
          pltpu.SemaphoreType.DMA,
          pltpu.SemaphoreType.DMA,
      ],
      compiler_params=pltpu.CompilerParams(use_tc_tiling_on_sc=False,
                                           needs_layout_passes=False),
  )
  def gather_k(i_emb_h, o_emb_h, ctx_ids_h, tgt_ids_h, neg_ids_h,
               ctx_out, tgt_out, neg_out,
               i0, i1, buf0, buf1, sem0, sem1):
    wid = lax.axis_index("s") * NC + lax.axis_index("c")
    base = wid * spw
    ibufs = (i0, i1)
    bufs = (buf0, buf1)
    sems = (sem0, sem1)

    def build_idx(ids_h, j, ibuf):
      pltpu.sync_copy(ids_h.at[pl.ds(j * B + base, spw)], ibuf)

    def run(tab_h, n_slots, build, scatter):
      def fire(j, slot):
        build(j, ibufs[slot])
        pltpu.async_copy(tab_h.at[ibufs[slot]], bufs[slot], sems[slot])

      def drain(j, slot):
        pltpu.make_async_copy(tab_h.at[ibufs[slot]], bufs[slot],
                              sems[slot]).wait()
        scatter(j, bufs[slot])

      if n_slots == 1:
        fire(0, 0)
        drain(0, 0)
        return

      fire(0, 0)

      def body(k, carry):
        j0 = k * 2
        fire(j0 + 1, 1)
        drain(j0, 0)

        @pl.when(k < n_slots // 2 - 1)
        def _():
          fire(j0 + 2, 0)

        drain(j0 + 1, 1)
        return carry

      lax.fori_loop(0, n_slots // 2, body, 0)

    def scatter_pair(out_h):
      def scatter(j, buf):
        row0 = (j // 2) * B + base
        col0 = (j % 2) * D
        pltpu.sync_copy(buf, out_h.at[pl.ds(row0, spw), pl.ds(col0, D)])
      return scatter

    run(i_emb_h, CTX,
        lambda j, ibuf: build_idx(ctx_ids_h, j, ibuf),
        scatter_pair(ctx_out))
    run(o_emb_h, 1,
        lambda j, ibuf: pltpu.sync_copy(tgt_ids_h.at[pl.ds(base, spw)],
                                        ibuf),
        lambda j, buf: pltpu.sync_copy(buf, tgt_out.at[pl.ds(base, spw)]))
    run(o_emb_h, NNEG,
        lambda j, ibuf: build_idx(neg_ids_h, j, ibuf),
        scatter_pair(neg_out))

  return gather_k(i_emb, o_emb, ctx_ids, tgt_ids, neg_ids)


def _dense_body(ctx_ref, tgt_ref, neg_ref, W1_ref, b1_ref, W2_ref, b2_ref,
                seg_ref, out_ref):
  f32 = jnp.float32

  hp = jnp.dot(ctx_ref[0], W1_ref[pl.ds(0, 2 * D), :],
               preferred_element_type=f32)
  for jp in range(1, CTX // 2):
    hp = hp + jnp.dot(ctx_ref[jp], W1_ref[pl.ds(jp * 2 * D, 2 * D), :],
                      preferred_element_type=f32)
  h = jnp.tanh(hp + b1_ref[...])
  logits = jnp.dot(h, W2_ref[...],
                   preferred_element_type=f32) + b2_ref[...]
  a = jax.nn.softmax(logits, axis=-1)

  attn128 = jnp.zeros(hp.shape[:1] + (2 * D,), f32)
  lane128 = lax.broadcasted_iota(jnp.int32, hp.shape[:1] + (2 * D,), 1)
  for jp in range(CTX // 2):
    aw = jnp.where(lane128 < D, a[:, 2 * jp:2 * jp + 1],
                   a[:, 2 * jp + 1:2 * jp + 2])
    attn128 = attn128 + aw * ctx_ref[jp]
  attn = attn128[:, 0:D] + attn128[:, D:2 * D]

  seg = seg_ref[...]
  pos_dot = jnp.sum(tgt_ref[...] * attn, axis=1, keepdims=True)
  attn2 = jnp.concatenate([attn, attn], axis=1)
  nds = [jnp.dot(neg_ref[k] * attn2, seg,
                 preferred_element_type=f32)[:, 0:2]
         for k in range(NNEG // 2)]
  all_dots = jnp.concatenate([pos_dot] + [-n for n in nds], axis=1)

  acc = jnp.sum(jnp.log(jax.nn.sigmoid(all_dots)))

  @pl.when(pl.program_id(0) == 0)
  def _():
    out_ref[0, 0] = 0.0

  out_ref[0, 0] += acc


def _seg_matrix():
  rows = jnp.arange(2 * D)[:, None]
  cols = jnp.arange(2 * D)[None, :]
  return jnp.where(cols == 0, rows < D,
                   jnp.where(cols == 1, rows >= D, False)
                   ).astype(jnp.float32)


def kernel(target_wids, context_wids, neg_wids, i_emb, o_emb, W1, b1, W2, b2):
  B = target_wids.shape[0]
  ctx_ids = context_wids.astype(jnp.int32).T.reshape(-1)
  tgt_ids = target_wids.astype(jnp.int32)
  neg_ids = neg_wids.astype(jnp.int32).T.reshape(-1)

  ctx_pair, tgt, neg_pair = _sc_gather(i_emb, o_emb, ctx_ids, tgt_ids,
                                       neg_ids)
  ctx3 = ctx_pair.reshape(CTX // 2, B, 2 * D)
  neg3 = neg_pair.reshape(NNEG // 2, B, 2 * D)

  bB = 1024
  grid = B // bB
  loss = pl.pallas_call(
      _dense_body,
      grid=(grid,),
      in_specs=[
          pl.BlockSpec((CTX // 2, bB, 2 * D), lambda i: (0, i, 0)),
          pl.BlockSpec((bB, D), lambda i: (i, 0)),
          pl.BlockSpec((NNEG // 2, bB, 2 * D), lambda i: (0, i, 0)),
          pl.BlockSpec((CTX * D, 50), lambda i: (0, 0)),
          pl.BlockSpec((1, 50), lambda i: (0, 0)),
          pl.BlockSpec((50, CTX), lambda i: (0, 0)),
          pl.BlockSpec((1, CTX), lambda i: (0, 0)),
          pl.BlockSpec((2 * D, 2 * D), lambda i: (0, 0)),
      ],
      out_specs=pl.BlockSpec((1, 1), lambda i: (0, 0),
                             memory_space=pltpu.SMEM),
      out_shape=jax.ShapeDtypeStruct((1, 1), jnp.float32),
  )(ctx3, tgt, neg3, W1, b1.reshape(1, 50), W2, b2.reshape(1, CTX),
    _seg_matrix())

  return -loss[0, 0]


# speedup vs baseline: 1.2438x; 1.0241x over previous
import functools

import jax
import jax.numpy as jnp
from jax import lax
from jax.experimental import pallas as pl
from jax.experimental.pallas import tpu as pltpu
from jax.experimental.pallas import tpu_sc as plsc

D = 64
CTX = 10
NNEG = 20

NC = 2
NS = 16
NW = NC * NS
L = 16


def _sc_gather(i_emb, o_emb, ctx_ids, tgt_ids, neg_ids):
  B = tgt_ids.shape[0]
  spw = B // NW

  mesh = plsc.VectorSubcoreMesh(core_axis_name="c", subcore_axis_name="s")

  @functools.partial(
      pl.kernel,
      mesh=mesh,
      out_type=[
          jax.ShapeDtypeStruct((CTX // 2 * B, 2 * D), jnp.float32),
          jax.ShapeDtypeStruct((B, D), jnp.float32),
          jax.ShapeDtypeStruct((NNEG // 2 * B, 2 * D), jnp.float32),
      ],
      scratch_types=[
          pltpu.VMEM((spw,), jnp.int32),
          pltpu.VMEM((spw,), jnp.int32),
          pltpu.VMEM((spw, D), jnp.float32),
          pltpu.VMEM((spw, D), jnp.float32),
          pltpu.SemaphoreType.DMA,
          pltpu.SemaphoreType.DMA,
      ],
      compiler_params=pltpu.CompilerParams(use_tc_tiling_on_sc=False,
                                           needs_layout_passes=False),
  )
  def gather_k(i_emb_h, o_emb_h, ctx_ids_h, tgt_ids_h, neg_ids_h,
               ctx_out, tgt_out, neg_out,
               i0, i1, buf0, buf1, sem0, sem1):
    wid = lax.axis_index("s") * NC + lax.axis_index("c")
    base = wid * spw
    ibufs = (i0, i1)
    bufs = (buf0, buf1)
    sems = (sem0, sem1)

    def build_idx(ids_h, j, ibuf):
      pltpu.sync_copy(ids_h.at[pl.ds(j * B + base, spw)], ibuf)

    def run(tab_h, n_slots, build, scatter):
      def fire(j, slot):
        build(j, ibufs[slot])
        pltpu.async_copy(tab_h.at[ibufs[slot]], bufs[slot], sems[slot])

      def drain(j, slot):
        pltpu.make_async_copy(tab_h.at[ibufs[slot]], bufs[slot],
                              sems[slot]).wait()
        scatter(j, bufs[slot])

      if n_slots == 1:
        fire(0, 0)
        drain(0, 0)
        return

      fire(0, 0)

      def body(k, carry):
        j0 = k * 2
        fire(j0 + 1, 1)
        drain(j0, 0)

        @pl.when(k < n_slots // 2 - 1)
        def _():
          fire(j0 + 2, 0)

        drain(j0 + 1, 1)
        return carry

      lax.fori_loop(0, n_slots // 2, body, 0)

    def scatter_pair(out_h):
      def scatter(j, buf):
        row0 = (j // 2) * B + base
        col0 = (j % 2) * D
        pltpu.sync_copy(buf, out_h.at[pl.ds(row0, spw), pl.ds(col0, D)])
      return scatter

    run(i_emb_h, CTX,
        lambda j, ibuf: build_idx(ctx_ids_h, j, ibuf),
        scatter_pair(ctx_out))
    run(o_emb_h, 1,
        lambda j, ibuf: pltpu.sync_copy(tgt_ids_h.at[pl.ds(base, spw)],
                                        ibuf),
        lambda j, buf: pltpu.sync_copy(buf, tgt_out.at[pl.ds(base, spw)]))
    run(o_emb_h, NNEG,
        lambda j, ibuf: build_idx(neg_ids_h, j, ibuf),
        scatter_pair(neg_out))

  return gather_k(i_emb, o_emb, ctx_ids, tgt_ids, neg_ids)


def _dense_body(ctx_ref, tgt_ref, neg_ref, W1_ref, b1_ref, W2_ref, b2_ref,
                seg_ref, e_ref, out_ref):
  f32 = jnp.float32

  hp = jnp.dot(ctx_ref[0], W1_ref[pl.ds(0, 2 * D), :],
               preferred_element_type=f32)
  for jp in range(1, CTX // 2):
    hp = hp + jnp.dot(ctx_ref[jp], W1_ref[pl.ds(jp * 2 * D, 2 * D), :],
                      preferred_element_type=f32)
  h = jnp.tanh(hp + b1_ref[...])
  logits = jnp.dot(h, W2_ref[...],
                   preferred_element_type=f32) + b2_ref[...]
  a = jax.nn.softmax(logits, axis=-1)

  a128 = jnp.concatenate(
      [a, jnp.zeros(a.shape[:1] + (2 * D - CTX,), f32)], axis=1)
  attn128 = jnp.zeros(hp.shape[:1] + (2 * D,), f32)
  for jp in range(CTX // 2):
    aw = jnp.dot(a128, e_ref[:, pl.ds(jp * 2 * D, 2 * D)],
                 preferred_element_type=f32)
    attn128 = attn128 + aw * ctx_ref[jp]
  attn = attn128[:, 0:D] + attn128[:, D:2 * D]

  seg = seg_ref[...]
  pos_dot = jnp.sum(tgt_ref[...] * attn, axis=1, keepdims=True)
  attn2 = jnp.concatenate([attn, attn], axis=1)
  nds = [jnp.dot(neg_ref[k] * attn2, seg,
                 preferred_element_type=f32)[:, 0:2]
         for k in range(NNEG // 2)]
  all_dots = jnp.concatenate([pos_dot] + [-n for n in nds], axis=1)

  acc = jnp.sum(jnp.log(jax.nn.sigmoid(all_dots)))

  @pl.when(pl.program_id(0) == 0)
  def _():
    out_ref[0, 0] = 0.0

  out_ref[0, 0] += acc


def _seg_matrix():
  rows = jnp.arange(2 * D)[:, None]
  cols = jnp.arange(2 * D)[None, :]
  return jnp.where(cols == 0, rows < D,
                   jnp.where(cols == 1, rows >= D, False)
                   ).astype(jnp.float32)


def _e_matrix():
  rows = jnp.arange(2 * D)[:, None]
  cols = jnp.arange(CTX // 2 * 2 * D)[None, :]
  jp = cols // (2 * D)
  lane = cols % (2 * D)
  return (rows == 2 * jp + (lane >= D)).astype(jnp.float32)


def kernel(target_wids, context_wids, neg_wids, i_emb, o_emb, W1, b1, W2, b2):
  B = target_wids.shape[0]
  ctx_ids = context_wids.astype(jnp.int32).T.reshape(-1)
  tgt_ids = target_wids.astype(jnp.int32)
  neg_ids = neg_wids.astype(jnp.int32).T.reshape(-1)

  ctx_pair, tgt, neg_pair = _sc_gather(i_emb, o_emb, ctx_ids, tgt_ids,
                                       neg_ids)
  ctx3 = ctx_pair.reshape(CTX // 2, B, 2 * D)
  neg3 = neg_pair.reshape(NNEG // 2, B, 2 * D)

  bB = 2048
  grid = B // bB
  loss = pl.pallas_call(
      _dense_body,
      grid=(grid,),
      in_specs=[
          pl.BlockSpec((CTX // 2, bB, 2 * D), lambda i: (0, i, 0)),
          pl.BlockSpec((bB, D), lambda i: (i, 0)),
          pl.BlockSpec((NNEG // 2, bB, 2 * D), lambda i: (0, i, 0)),
          pl.BlockSpec((CTX * D, 50), lambda i: (0, 0)),
          pl.BlockSpec((1, 50), lambda i: (0, 0)),
          pl.BlockSpec((50, CTX), lambda i: (0, 0)),
          pl.BlockSpec((1, CTX), lambda i: (0, 0)),
          pl.BlockSpec((2 * D, 2 * D), lambda i: (0, 0)),
          pl.BlockSpec((2 * D, CTX // 2 * 2 * D), lambda i: (0, 0)),
      ],
      out_specs=pl.BlockSpec((1, 1), lambda i: (0, 0),
                             memory_space=pltpu.SMEM),
      out_shape=jax.ShapeDtypeStruct((1, 1), jnp.float32),
  )(ctx3, tgt, neg3, W1, b1.reshape(1, 50), W2, b2.reshape(1, CTX),
    _seg_matrix(), _e_matrix())

  return -loss[0, 0]


# speedup vs baseline: 1.4045x; 1.1292x over previous
import functools

import jax
import jax.numpy as jnp
from jax import lax
from jax.experimental import pallas as pl
from jax.experimental.pallas import tpu as pltpu
from jax.experimental.pallas import tpu_sc as plsc

D = 64
CTX = 10
NNEG = 20

NC = 2
NS = 16
NW = NC * NS
L = 16


def _worker_gather(B, spw, tab_h, base, ibufs, bufs, sems, n_slots,
                   build, scatter):
  def fire(j, slot):
    build(j, ibufs[slot])
    pltpu.async_copy(tab_h.at[ibufs[slot]], bufs[slot], sems[slot])

  def drain(j, slot):
    pltpu.make_async_copy(tab_h.at[ibufs[slot]], bufs[slot],
                          sems[slot]).wait()
    scatter(j, bufs[slot])

  if n_slots == 1:
    fire(0, 0)
    drain(0, 0)
    return

  fire(0, 0)

  def body(k, carry):
    j0 = k * 2
    fire(j0 + 1, 1)
    drain(j0, 0)

    @pl.when(k < n_slots // 2 - 1)
    def _():
      fire(j0 + 2, 0)

    drain(j0 + 1, 1)
    return carry

  lax.fori_loop(0, n_slots // 2, body, 0)


def _gather_scratch(spw):
  return [
      pltpu.VMEM((spw,), jnp.int32),
      pltpu.VMEM((spw,), jnp.int32),
      pltpu.VMEM((spw, D), jnp.float32),
      pltpu.VMEM((spw, D), jnp.float32),
      pltpu.SemaphoreType.DMA,
      pltpu.SemaphoreType.DMA,
  ]


_SC_PARAMS = dict(
    compiler_params=pltpu.CompilerParams(use_tc_tiling_on_sc=False,
                                         needs_layout_passes=False))


def _sc_gather_ctx(i_emb, ctx_ids):
  B = ctx_ids.shape[0] // CTX
  spw = B // NW

  mesh = plsc.VectorSubcoreMesh(core_axis_name="c", subcore_axis_name="s")

  @functools.partial(
      pl.kernel, mesh=mesh,
      out_type=jax.ShapeDtypeStruct((CTX // 2 * B, 2 * D), jnp.float32),
      scratch_types=_gather_scratch(spw), **_SC_PARAMS)
  def gather_k(i_emb_h, ctx_ids_h, ctx_out, i0, i1, buf0, buf1, sem0, sem1):
    wid = lax.axis_index("s") * NC + lax.axis_index("c")
    base = wid * spw

    def build(j, ibuf):
      pltpu.sync_copy(ctx_ids_h.at[pl.ds(j * B + base, spw)], ibuf)

    def scatter(j, buf):
      pltpu.sync_copy(buf, ctx_out.at[pl.ds((j // 2) * B + base, spw),
                                      pl.ds((j % 2) * D, D)])

    _worker_gather(B, spw, i_emb_h, base, (i0, i1), (buf0, buf1),
                   (sem0, sem1), CTX, build, scatter)

  return gather_k(i_emb, ctx_ids)


def _sc_gather_tn(o_emb, tgt_ids, neg_ids):
  B = tgt_ids.shape[0]
  spw = B // NW

  mesh = plsc.VectorSubcoreMesh(core_axis_name="c", subcore_axis_name="s")

  @functools.partial(
      pl.kernel, mesh=mesh,
      out_type=[
          jax.ShapeDtypeStruct((B, D), jnp.float32),
          jax.ShapeDtypeStruct((NNEG // 2 * B, 2 * D), jnp.float32),
      ],
      scratch_types=_gather_scratch(spw), **_SC_PARAMS)
  def gather_k(o_emb_h, tgt_ids_h, neg_ids_h, tgt_out, neg_out,
               i0, i1, buf0, buf1, sem0, sem1):
    wid = lax.axis_index("s") * NC + lax.axis_index("c")
    base = wid * spw
    ibufs = (i0, i1)
    bufs = (buf0, buf1)
    sems = (sem0, sem1)

    _worker_gather(
        B, spw, o_emb_h, base, ibufs, bufs, sems, 1,
        lambda j, ibuf: pltpu.sync_copy(tgt_ids_h.at[pl.ds(base, spw)],
                                        ibuf),
        lambda j, buf: pltpu.sync_copy(buf, tgt_out.at[pl.ds(base, spw)]))
    _worker_gather(
        B, spw, o_emb_h, base, ibufs, bufs, sems, NNEG,
        lambda j, ibuf: pltpu.sync_copy(
            neg_ids_h.at[pl.ds(j * B + base, spw)], ibuf),
        lambda j, buf: pltpu.sync_copy(
            buf, neg_out.at[pl.ds((j // 2) * B + base, spw),
                            pl.ds((j % 2) * D, D)]))

  return gather_k(o_emb, tgt_ids, neg_ids)


def _attn_body(ctx_ref, W1_ref, b1_ref, W2_ref, b2_ref, e_ref, attn2_ref):
  f32 = jnp.float32

  hp = jnp.dot(ctx_ref[0], W1_ref[pl.ds(0, 2 * D), :],
               preferred_element_type=f32)
  for jp in range(1, CTX // 2):
    hp = hp + jnp.dot(ctx_ref[jp], W1_ref[pl.ds(jp * 2 * D, 2 * D), :],
                      preferred_element_type=f32)
  h = jnp.tanh(hp + b1_ref[...])
  logits = jnp.dot(h, W2_ref[...],
                   preferred_element_type=f32) + b2_ref[...]
  a = jax.nn.softmax(logits, axis=-1)

  a128 = jnp.concatenate(
      [a, jnp.zeros(a.shape[:1] + (2 * D - CTX,), f32)], axis=1)
  attn128 = jnp.zeros(hp.shape[:1] + (2 * D,), f32)
  for jp in range(CTX // 2):
    aw = jnp.dot(a128, e_ref[:, pl.ds(jp * 2 * D, 2 * D)],
                 preferred_element_type=f32)
    attn128 = attn128 + aw * ctx_ref[jp]
  attn = attn128[:, 0:D] + attn128[:, D:2 * D]
  attn2_ref[...] = jnp.concatenate([attn, attn], axis=1)


def _loss_body(tgt_ref, neg_ref, attn2_ref, seg_ref, out_ref):
  f32 = jnp.float32
  attn2 = attn2_ref[...]

  seg = seg_ref[...]
  pos_dot = jnp.sum(tgt_ref[...] * attn2[:, 0:D], axis=1,
                    keepdims=True)
  nds = [jnp.dot(neg_ref[k] * attn2, seg,
                 preferred_element_type=f32)[:, 0:2]
         for k in range(NNEG // 2)]
  all_dots = jnp.concatenate([pos_dot] + [-n for n in nds], axis=1)

  acc = jnp.sum(jnp.log(jax.nn.sigmoid(all_dots)))

  @pl.when(pl.program_id(0) == 0)
  def _():
    out_ref[0, 0] = 0.0

  out_ref[0, 0] += acc


def _seg_matrix():
  rows = jnp.arange(2 * D)[:, None]
  cols = jnp.arange(2 * D)[None, :]
  return jnp.where(cols == 0, rows < D,
                   jnp.where(cols == 1, rows >= D, False)
                   ).astype(jnp.float32)


def _e_matrix():
  rows = jnp.arange(2 * D)[:, None]
  cols = jnp.arange(CTX // 2 * 2 * D)[None, :]
  jp = cols // (2 * D)
  lane = cols % (2 * D)
  return (rows == 2 * jp + (lane >= D)).astype(jnp.float32)


def kernel(target_wids, context_wids, neg_wids, i_emb, o_emb, W1, b1, W2, b2):
  B = target_wids.shape[0]
  ctx_ids = context_wids.astype(jnp.int32).T.reshape(-1)
  tgt_ids = target_wids.astype(jnp.int32)
  neg_ids = neg_wids.astype(jnp.int32).T.reshape(-1)

  ctx_pair = _sc_gather_ctx(i_emb, ctx_ids)
  ctx3 = ctx_pair.reshape(CTX // 2, B, 2 * D)
  tgt, neg_pair = _sc_gather_tn(o_emb, tgt_ids, neg_ids)
  neg3 = neg_pair.reshape(NNEG // 2, B, 2 * D)

  bB = 2048
  grid = B // bB
  attn2 = pl.pallas_call(
      _attn_body,
      grid=(grid,),
      in_specs=[
          pl.BlockSpec((CTX // 2, bB, 2 * D), lambda i: (0, i, 0)),
          pl.BlockSpec((CTX * D, 50), lambda i: (0, 0)),
          pl.BlockSpec((1, 50), lambda i: (0, 0)),
          pl.BlockSpec((50, CTX), lambda i: (0, 0)),
          pl.BlockSpec((1, CTX), lambda i: (0, 0)),
          pl.BlockSpec((2 * D, CTX // 2 * 2 * D), lambda i: (0, 0)),
      ],
      out_specs=pl.BlockSpec((bB, 2 * D), lambda i: (i, 0)),
      out_shape=jax.ShapeDtypeStruct((B, 2 * D), jnp.float32),
  )(ctx3, W1, b1.reshape(1, 50), W2, b2.reshape(1, CTX), _e_matrix())

  loss = pl.pallas_call(
      _loss_body,
      grid=(grid,),
      in_specs=[
          pl.BlockSpec((bB, D), lambda i: (i, 0)),
          pl.BlockSpec((NNEG // 2, bB, 2 * D), lambda i: (0, i, 0)),
          pl.BlockSpec((bB, 2 * D), lambda i: (i, 0)),
          pl.BlockSpec((2 * D, 2 * D), lambda i: (0, 0)),
      ],
      out_specs=pl.BlockSpec((1, 1), lambda i: (0, 0),
                             memory_space=pltpu.SMEM),
      out_shape=jax.ShapeDtypeStruct((1, 1), jnp.float32),
  )(tgt, neg3, attn2, _seg_matrix())

  return -loss[0, 0]


# speedup vs baseline: 1.4397x; 1.0251x over previous
import functools

import jax
import jax.numpy as jnp
from jax import lax
from jax.experimental import pallas as pl
from jax.experimental.pallas import tpu as pltpu
from jax.experimental.pallas import tpu_sc as plsc

D = 64
CTX = 10
NNEG = 20

NC = 2
NS = 16
NW = NC * NS
L = 16


def _worker_gather(B, spw, tab_h, base, ibufs, bufs, sems, n_slots,
                   build, scatter):
  def fire(j, slot):
    build(j, ibufs[slot])
    pltpu.async_copy(tab_h.at[ibufs[slot]], bufs[slot], sems[slot])

  def drain(j, slot):
    pltpu.make_async_copy(tab_h.at[ibufs[slot]], bufs[slot],
                          sems[slot]).wait()
    scatter(j, bufs[slot])

  if n_slots == 1:
    fire(0, 0)
    drain(0, 0)
    return

  fire(0, 0)

  def body(k, carry):
    j0 = k * 2
    fire(j0 + 1, 1)
    drain(j0, 0)

    @pl.when(k < n_slots // 2 - 1)
    def _():
      fire(j0 + 2, 0)

    drain(j0 + 1, 1)
    return carry

  lax.fori_loop(0, n_slots // 2, body, 0)


def _gather_scratch(spw):
  return [
      pltpu.VMEM((spw,), jnp.int32),
      pltpu.VMEM((spw,), jnp.int32),
      pltpu.VMEM((spw, D), jnp.float32),
      pltpu.VMEM((spw, D), jnp.float32),
      pltpu.SemaphoreType.DMA,
      pltpu.SemaphoreType.DMA,
  ]


_SC_PARAMS = dict(
    compiler_params=pltpu.CompilerParams(use_tc_tiling_on_sc=False,
                                         needs_layout_passes=False))


def _sc_gather_ctx(i_emb, ctx_ids):
  B = ctx_ids.shape[0] // CTX
  spw = B // NW

  mesh = plsc.VectorSubcoreMesh(core_axis_name="c", subcore_axis_name="s")

  @functools.partial(
      pl.kernel, mesh=mesh,
      out_type=jax.ShapeDtypeStruct((CTX // 2 * B, 2 * D), jnp.float32),
      scratch_types=_gather_scratch(spw), **_SC_PARAMS)
  def gather_k(i_emb_h, ctx_ids_h, ctx_out, i0, i1, buf0, buf1, sem0, sem1):
    wid = lax.axis_index("s") * NC + lax.axis_index("c")
    base = wid * spw

    def build(j, ibuf):
      pltpu.sync_copy(ctx_ids_h.at[pl.ds(j * B + base, spw)], ibuf)

    def scatter(j, buf):
      pltpu.sync_copy(buf, ctx_out.at[pl.ds((j // 2) * B + base, spw),
                                      pl.ds((j % 2) * D, D)])

    _worker_gather(B, spw, i_emb_h, base, (i0, i1), (buf0, buf1),
                   (sem0, sem1), CTX, build, scatter)

  return gather_k(i_emb, ctx_ids)


def _sc_gather_tn(o_emb, tgt_ids, neg_ids, dep):
  B = tgt_ids.shape[0]
  spw = B // NW

  mesh = plsc.VectorSubcoreMesh(core_axis_name="c", subcore_axis_name="s")

  @functools.partial(
      pl.kernel, mesh=mesh,
      out_type=[
          jax.ShapeDtypeStruct((B, D), jnp.float32),
          jax.ShapeDtypeStruct((NNEG // 2 * B, 2 * D), jnp.float32),
      ],
      scratch_types=_gather_scratch(spw), **_SC_PARAMS)
  def gather_k(o_emb_h, tgt_ids_h, neg_ids_h, dep_h, tgt_out, neg_out,
               i0, i1, buf0, buf1, sem0, sem1):
    del dep_h
    wid = lax.axis_index("s") * NC + lax.axis_index("c")
    base = wid * spw
    ibufs = (i0, i1)
    bufs = (buf0, buf1)
    sems = (sem0, sem1)

    _worker_gather(
        B, spw, o_emb_h, base, ibufs, bufs, sems, 1,
        lambda j, ibuf: pltpu.sync_copy(tgt_ids_h.at[pl.ds(base, spw)],
                                        ibuf),
        lambda j, buf: pltpu.sync_copy(buf, tgt_out.at[pl.ds(base, spw)]))
    _worker_gather(
        B, spw, o_emb_h, base, ibufs, bufs, sems, NNEG,
        lambda j, ibuf: pltpu.sync_copy(
            neg_ids_h.at[pl.ds(j * B + base, spw)], ibuf),
        lambda j, buf: pltpu.sync_copy(
            buf, neg_out.at[pl.ds((j // 2) * B + base, spw),
                            pl.ds((j % 2) * D, D)]))

  return gather_k(o_emb, tgt_ids, neg_ids, dep)


def _attn_body(ctx_ref, W1_ref, b1_ref, W2_ref, b2_ref, e_ref, attn2_ref):
  f32 = jnp.float32

  hp = jnp.dot(ctx_ref[0], W1_ref[pl.ds(0, 2 * D), :],
               preferred_element_type=f32)
  for jp in range(1, CTX // 2):
    hp = hp + jnp.dot(ctx_ref[jp], W1_ref[pl.ds(jp * 2 * D, 2 * D), :],
                      preferred_element_type=f32)
  h = jnp.tanh(hp + b1_ref[...])
  logits = jnp.dot(h, W2_ref[...],
                   preferred_element_type=f32) + b2_ref[...]
  a = jax.nn.softmax(logits, axis=-1)

  a128 = jnp.concatenate(
      [a, jnp.zeros(a.shape[:1] + (2 * D - CTX,), f32)], axis=1)
  attn128 = jnp.zeros(hp.shape[:1] + (2 * D,), f32)
  for jp in range(CTX // 2):
    aw = jnp.dot(a128, e_ref[:, pl.ds(jp * 2 * D, 2 * D)],
                 preferred_element_type=f32)
    attn128 = attn128 + aw * ctx_ref[jp]
  attn = attn128[:, 0:D] + attn128[:, D:2 * D]
  attn2_ref[...] = jnp.concatenate([attn, attn], axis=1)


def _loss_body(tgt_ref, neg_ref, attn2_ref, seg_ref, out_ref):
  f32 = jnp.float32
  attn2 = attn2_ref[...]

  seg = seg_ref[...]
  pos_dot = jnp.sum(tgt_ref[...] * attn2[:, 0:D], axis=1,
                    keepdims=True)
  nds = [jnp.dot(neg_ref[k] * attn2, seg,
                 preferred_element_type=f32)[:, 0:2]
         for k in range(NNEG // 2)]
  all_dots = jnp.concatenate([pos_dot] + [-n for n in nds], axis=1)

  acc = jnp.sum(jnp.log(jax.nn.sigmoid(all_dots)))

  @pl.when(pl.program_id(0) == 0)
  def _():
    out_ref[0, 0] = 0.0

  out_ref[0, 0] += acc


def _seg_matrix():
  rows = jnp.arange(2 * D)[:, None]
  cols = jnp.arange(2 * D)[None, :]
  return jnp.where(cols == 0, rows < D,
                   jnp.where(cols == 1, rows >= D, False)
                   ).astype(jnp.float32)


def _e_matrix():
  rows = jnp.arange(2 * D)[:, None]
  cols = jnp.arange(CTX // 2 * 2 * D)[None, :]
  jp = cols // (2 * D)
  lane = cols % (2 * D)
  return (rows == 2 * jp + (lane >= D)).astype(jnp.float32)


def kernel(target_wids, context_wids, neg_wids, i_emb, o_emb, W1, b1, W2, b2):
  B = target_wids.shape[0]
  ctx_ids = context_wids.astype(jnp.int32).T.reshape(-1)
  tgt_ids = target_wids.astype(jnp.int32)
  neg_ids = neg_wids.astype(jnp.int32).T.reshape(-1)

  ctx_pair = _sc_gather_ctx(i_emb, ctx_ids)
  ctx3 = ctx_pair.reshape(CTX // 2, B, 2 * D)
  tgt, neg_pair = _sc_gather_tn(o_emb, tgt_ids, neg_ids,
                                ctx_pair[:1])
  neg3 = neg_pair.reshape(NNEG // 2, B, 2 * D)

  bB = 2048
  grid = B // bB
  attn2 = pl.pallas_call(
      _attn_body,
      grid=(grid,),
      in_specs=[
          pl.BlockSpec((CTX // 2, bB, 2 * D), lambda i: (0, i, 0)),
          pl.BlockSpec((CTX * D, 50), lambda i: (0, 0)),
          pl.BlockSpec((1, 50), lambda i: (0, 0)),
          pl.BlockSpec((50, CTX), lambda i: (0, 0)),
          pl.BlockSpec((1, CTX), lambda i: (0, 0)),
          pl.BlockSpec((2 * D, CTX // 2 * 2 * D), lambda i: (0, 0)),
      ],
      out_specs=pl.BlockSpec((bB, 2 * D), lambda i: (i, 0)),
      out_shape=jax.ShapeDtypeStruct((B, 2 * D), jnp.float32),
  )(ctx3, W1, b1.reshape(1, 50), W2, b2.reshape(1, CTX), _e_matrix())

  loss = pl.pallas_call(
      _loss_body,
      grid=(grid,),
      in_specs=[
          pl.BlockSpec((bB, D), lambda i: (i, 0)),
          pl.BlockSpec((NNEG // 2, bB, 2 * D), lambda i: (0, i, 0)),
          pl.BlockSpec((bB, 2 * D), lambda i: (i, 0)),
          pl.BlockSpec((2 * D, 2 * D), lambda i: (0, 0)),
      ],
      out_specs=pl.BlockSpec((1, 1), lambda i: (0, 0),
                             memory_space=pltpu.SMEM),
      out_shape=jax.ShapeDtypeStruct((1, 1), jnp.float32),
  )(tgt, neg3, attn2, _seg_matrix())

  return -loss[0, 0]
